# baseline (speedup 1.0000x reference)
import functools

import jax
import jax.numpy as jnp
from jax import lax
from jax.experimental import pallas as pl
from jax.experimental.pallas import tpu as pltpu
from jax.experimental.pallas import tpu_sc as plsc


def _routing_body(x_ref, wr_ref, br_ref, pos_ref, off_ref, blk_ref,
                  eid_ref, *, T, G):
    S, _ = x_ref.shape
    E = wr_ref.shape[0]
    log2t = T.bit_length() - 1

    x = x_ref[...]
    wr = wr_ref[...]
    logits = lax.dot_general(x, wr, (((1,), (1,)), ((), ())),
                             preferred_element_type=jnp.float32)
    logits = logits + br_ref[...]
    eid = jnp.argmax(logits, axis=1).astype(jnp.int32)[:, None]

    iota_se = lax.broadcasted_iota(jnp.int32, (S, E), 1)
    onehot = (iota_se == eid).astype(jnp.float32)
    counts_row = jnp.sum(onehot, axis=0, keepdims=True)

    r_i = lax.broadcasted_iota(jnp.int32, (S, S), 0)
    c_i = lax.broadcasted_iota(jnp.int32, (S, S), 1)
    lt = (c_i < r_i).astype(jnp.float32)
    cum = lax.dot_general(lt, onehot, (((1,), (0,)), ((), ())),
                          preferred_element_type=jnp.float32)
    rank = jnp.sum(cum * onehot, axis=1, keepdims=True)

    r_e = lax.broadcasted_iota(jnp.int32, (E, E), 0)
    c_e = lax.broadcasted_iota(jnp.int32, (E, E), 1)
    m_excl = (r_e < c_e).astype(jnp.float32)
    m_incl = (r_e <= c_e).astype(jnp.float32)
    off_lo = lax.dot_general(counts_row, m_excl, (((1,), (0,)), ((), ())),
                             preferred_element_type=jnp.float32)
    off_hi = lax.dot_general(counts_row, m_incl, (((1,), (0,)), ((), ())),
                             preferred_element_type=jnp.float32)

    pos_off = jnp.sum(onehot * off_lo, axis=1, keepdims=True)
    pos = (pos_off + rank).astype(jnp.int32)
    pos_ref[...] = pos
    off_ref[...] = jnp.concatenate(
        [jnp.zeros((1, 1), jnp.float32), off_hi], axis=1).astype(jnp.int32)

    off_lo_i = off_lo.astype(jnp.int32)
    off_hi_i = off_hi.astype(jnp.int32)
    counts_i = off_hi_i - off_lo_i
    first_blk = lax.shift_right_arithmetic(off_lo_i, log2t)
    last_blk = lax.shift_right_arithmetic(off_hi_i - 1, log2t)
    nb = jnp.where(counts_i > 0, last_blk - first_blk + 1, 0)
    nb_f = nb.astype(jnp.float32)
    cum_nb = lax.dot_general(nb_f, m_incl, (((1,), (0,)), ((), ())),
                             preferred_element_type=jnp.float32)
    pair_start = cum_nb - nb_f
    p_total = jnp.sum(nb)

    g_iota = lax.broadcasted_iota(jnp.int32, (G, 1), 0)
    gg = jnp.minimum(g_iota, p_total - 1)
    gg_f = gg.astype(jnp.float32)
    e_of_g = jnp.sum((cum_nb <= gg_f).astype(jnp.int32), axis=1,
                     keepdims=True)
    iota_ge = lax.broadcasted_iota(jnp.int32, (G, E), 1)
    onehot_g = (iota_ge == e_of_g).astype(jnp.float32)
    firstblk_g = jnp.sum(onehot_g * first_blk.astype(jnp.float32), axis=1,
                         keepdims=True)
    pstart_g = jnp.sum(onehot_g * pair_start, axis=1, keepdims=True)
    blk_g = firstblk_g.astype(jnp.int32) + gg - pstart_g.astype(jnp.int32)
    blk_ref[...] = blk_g
    eid_ref[...] = e_of_g


def _sc_permute_rows(src, idx, *, invert):
    N, Dm = src.shape
    info = plsc.get_sparse_core_info()
    nc = info.num_cores
    nw = nc * info.num_subcores
    rpw = N // nw
    mesh = plsc.VectorSubcoreMesh(core_axis_name="c", subcore_axis_name="s")

    @functools.partial(
        pl.kernel, mesh=mesh,
        out_type=jax.ShapeDtypeStruct((N, Dm), jnp.float32),
        scratch_types=[
            pltpu.VMEM((rpw,), jnp.int32),
            pltpu.VMEM((rpw, Dm), jnp.float32),
            pltpu.SemaphoreType.DMA,
        ],
    )
    def k(src_hbm, idx_hbm, out_hbm, idx_v, rows_v, sem):
        wid = lax.axis_index("s") * nc + lax.axis_index("c")
        base = wid * rpw
        pltpu.sync_copy(idx_hbm.at[pl.ds(base, rpw)], idx_v)
        if invert:
            pltpu.async_copy(src_hbm.at[idx_v], rows_v, sem).wait()
            pltpu.sync_copy(rows_v, out_hbm.at[pl.ds(base, rpw)])
        else:
            pltpu.sync_copy(src_hbm.at[pl.ds(base, rpw)], rows_v)
            pltpu.async_copy(rows_v, out_hbm.at[idx_v], sem).wait()

    return k(src, idx)


def _ffn_body(blk_s, eid_s, off_s, xs_ref, w1_ref, b1_ref, w2_ref, b2_ref,
              out_ref, *, T):
    g = pl.program_id(0)
    k = pl.program_id(1)
    e = eid_s[g]
    lo = off_s[e]
    hi = off_s[e + 1]
    blk = blk_s[g]

    x = xs_ref[...]
    h = lax.dot_general(x, w1_ref[0], (((1,), (1,)), ((), ())),
                        preferred_element_type=jnp.float32)
    h = h + b1_ref[0]
    h = 0.5 * h * (1.0 + lax.erf(h * 0.7071067811865476))
    y = lax.dot_general(h, w2_ref[0], (((1,), (1,)), ((), ())),
                        preferred_element_type=jnp.float32)

    rows = blk * T + lax.broadcasted_iota(jnp.int32, (T, 1), 0)
    mask = (rows >= lo) & (rows < hi)
    prev = jnp.where(k == 0, b2_ref[0], out_ref[...])
    out_ref[...] = jnp.where(mask, prev + y, out_ref[...])


def kernel(x, W1, b1, W2, b2, Wr, br):
    B, S, D = x.shape
    E, FF, _ = W1.shape
    N = B * S
    T = 64
    G = N // T + E

    x_flat = x.reshape(N, D)

    pos, off, blkg, eidg = pl.pallas_call(
        functools.partial(_routing_body, T=T, G=G),
        out_shape=(
            jax.ShapeDtypeStruct((N, 1), jnp.int32),
            jax.ShapeDtypeStruct((1, E + 1), jnp.int32),
            jax.ShapeDtypeStruct((G, 1), jnp.int32),
            jax.ShapeDtypeStruct((G, 1), jnp.int32),
        ),
    )(x_flat, Wr, br.reshape(1, E))

    pos_flat = pos.reshape(N)
    xs = _sc_permute_rows(x_flat, pos_flat, invert=False)

    KF = 1
    FK = FF // KF
    grid_spec = pltpu.PrefetchScalarGridSpec(
        num_scalar_prefetch=3,
        grid=(G, KF),
        in_specs=[
            pl.BlockSpec((T, D), lambda g, k, blk, eid, off: (blk[g], 0)),
            pl.BlockSpec((1, FK, D),
                         lambda g, k, blk, eid, off: (eid[g], k, 0)),
            pl.BlockSpec((1, 1, FK),
                         lambda g, k, blk, eid, off: (eid[g], 0, k)),
            pl.BlockSpec((1, D, FK),
                         lambda g, k, blk, eid, off: (eid[g], 0, k)),
            pl.BlockSpec((1, 1, D),
                         lambda g, k, blk, eid, off: (eid[g], 0, 0)),
        ],
        out_specs=pl.BlockSpec((T, D), lambda g, k, blk, eid, off: (blk[g], 0)),
    )
    ys = pl.pallas_call(
        functools.partial(_ffn_body, T=T),
        grid_spec=grid_spec,
        out_shape=jax.ShapeDtypeStruct((N, D), jnp.float32),
        compiler_params=pltpu.CompilerParams(
            dimension_semantics=("arbitrary", "arbitrary")),
    )(blkg.reshape(G), eidg.reshape(G), off.reshape(E + 1),
      xs, W1, b1.reshape(E, 1, FF), W2, b2.reshape(E, 1, D))

    out = _sc_permute_rows(ys, pos_flat, invert=True)

    return out.reshape(B, S, D)


# speedup vs baseline: 1.2390x; 1.2390x over previous
import functools

import jax
import jax.numpy as jnp
from jax import lax
from jax.experimental import pallas as pl
from jax.experimental.pallas import tpu as pltpu
from jax.experimental.pallas import tpu_sc as plsc


def _routing_body(x_ref, wr_ref, br_ref, pos_ref, off_ref, blk_ref,
                  eid_ref, *, T, G):
    S, _ = x_ref.shape
    E = wr_ref.shape[0]
    log2t = T.bit_length() - 1

    x = x_ref[...]
    wr = wr_ref[...]
    logits = lax.dot_general(x, wr, (((1,), (1,)), ((), ())),
                             preferred_element_type=jnp.float32)
    logits = logits + br_ref[...]
    eid = jnp.argmax(logits, axis=1).astype(jnp.int32)[:, None]

    iota_se = lax.broadcasted_iota(jnp.int32, (S, E), 1)
    onehot = (iota_se == eid).astype(jnp.float32)
    counts_row = jnp.sum(onehot, axis=0, keepdims=True)

    r_i = lax.broadcasted_iota(jnp.int32, (S, S), 0)
    c_i = lax.broadcasted_iota(jnp.int32, (S, S), 1)
    lt = (c_i < r_i).astype(jnp.float32)
    cum = lax.dot_general(lt, onehot, (((1,), (0,)), ((), ())),
                          preferred_element_type=jnp.float32)
    rank = jnp.sum(cum * onehot, axis=1, keepdims=True)

    r_e = lax.broadcasted_iota(jnp.int32, (E, E), 0)
    c_e = lax.broadcasted_iota(jnp.int32, (E, E), 1)
    m_excl = (r_e < c_e).astype(jnp.float32)
    m_incl = (r_e <= c_e).astype(jnp.float32)
    off_lo = lax.dot_general(counts_row, m_excl, (((1,), (0,)), ((), ())),
                             preferred_element_type=jnp.float32)
    off_hi = lax.dot_general(counts_row, m_incl, (((1,), (0,)), ((), ())),
                             preferred_element_type=jnp.float32)

    pos_off = jnp.sum(onehot * off_lo, axis=1, keepdims=True)
    pos = (pos_off + rank).astype(jnp.int32)
    pos_ref[...] = pos
    off_ref[...] = jnp.concatenate(
        [jnp.zeros((1, 1), jnp.float32), off_hi], axis=1).astype(jnp.int32)

    off_lo_i = off_lo.astype(jnp.int32)
    off_hi_i = off_hi.astype(jnp.int32)
    counts_i = off_hi_i - off_lo_i
    first_blk = lax.shift_right_arithmetic(off_lo_i, log2t)
    last_blk = lax.shift_right_arithmetic(off_hi_i - 1, log2t)
    nb = jnp.where(counts_i > 0, last_blk - first_blk + 1, 0)
    nb_f = nb.astype(jnp.float32)
    cum_nb = lax.dot_general(nb_f, m_incl, (((1,), (0,)), ((), ())),
                             preferred_element_type=jnp.float32)
    pair_start = cum_nb - nb_f
    p_total = jnp.sum(nb)

    g_iota = lax.broadcasted_iota(jnp.int32, (G, 1), 0)
    gg = jnp.minimum(g_iota, p_total - 1)
    gg_f = gg.astype(jnp.float32)
    e_of_g = jnp.sum((cum_nb <= gg_f).astype(jnp.int32), axis=1,
                     keepdims=True)
    iota_ge = lax.broadcasted_iota(jnp.int32, (G, E), 1)
    onehot_g = (iota_ge == e_of_g).astype(jnp.float32)
    firstblk_g = jnp.sum(onehot_g * first_blk.astype(jnp.float32), axis=1,
                         keepdims=True)
    pstart_g = jnp.sum(onehot_g * pair_start, axis=1, keepdims=True)
    blk_g = firstblk_g.astype(jnp.int32) + gg - pstart_g.astype(jnp.int32)
    blk_ref[...] = blk_g
    eid_ref[...] = e_of_g


def _sc_permute_rows(src, idx, *, invert):
    N, Dm = src.shape
    info = plsc.get_sparse_core_info()
    nc = info.num_cores
    nw = nc * info.num_subcores
    rpw = N // nw
    mesh = plsc.VectorSubcoreMesh(core_axis_name="c", subcore_axis_name="s")

    @functools.partial(
        pl.kernel, mesh=mesh,
        out_type=jax.ShapeDtypeStruct((N, Dm), jnp.float32),
        scratch_types=[
            pltpu.VMEM((rpw,), jnp.int32),
            pltpu.VMEM((rpw, Dm), jnp.float32),
            pltpu.SemaphoreType.DMA,
        ],
    )
    def k(src_hbm, idx_hbm, out_hbm, idx_v, rows_v, sem):
        wid = lax.axis_index("s") * nc + lax.axis_index("c")
        base = wid * rpw
        pltpu.sync_copy(idx_hbm.at[pl.ds(base, rpw)], idx_v)
        if invert:
            pltpu.async_copy(src_hbm.at[idx_v], rows_v, sem).wait()
            pltpu.sync_copy(rows_v, out_hbm.at[pl.ds(base, rpw)])
        else:
            pltpu.sync_copy(src_hbm.at[pl.ds(base, rpw)], rows_v)
            pltpu.async_copy(rows_v, out_hbm.at[idx_v], sem).wait()

    return k(src, idx)


def _ffn_body(blk_s, eid_s, off_s, xs_ref, w1_ref, b1_ref, w2_ref, b2_ref,
              out_ref, *, T):
    g = pl.program_id(0)
    k = pl.program_id(1)
    e = eid_s[g]
    lo = off_s[e]
    hi = off_s[e + 1]
    blk = blk_s[g]

    x = xs_ref[...]
    h = lax.dot_general(x, w1_ref[0], (((1,), (1,)), ((), ())),
                        preferred_element_type=jnp.float32)
    h = h + b1_ref[0]
    h = 0.5 * h * (1.0 + lax.erf(h * 0.7071067811865476))
    y = lax.dot_general(h, w2_ref[0], (((1,), (1,)), ((), ())),
                        preferred_element_type=jnp.float32)

    rows = blk * T + lax.broadcasted_iota(jnp.int32, (T, 1), 0)
    mask = (rows >= lo) & (rows < hi)
    prev = jnp.where(k == 0, b2_ref[0], out_ref[...])
    out_ref[...] = jnp.where(mask, prev + y, out_ref[...])


def kernel(x, W1, b1, W2, b2, Wr, br):
    B, S, D = x.shape
    E, FF, _ = W1.shape
    N = B * S
    T = 256
    G = N // T + E

    x_flat = x.reshape(N, D)

    pos, off, blkg, eidg = pl.pallas_call(
        functools.partial(_routing_body, T=T, G=G),
        out_shape=(
            jax.ShapeDtypeStruct((N, 1), jnp.int32),
            jax.ShapeDtypeStruct((1, E + 1), jnp.int32),
            jax.ShapeDtypeStruct((G, 1), jnp.int32),
            jax.ShapeDtypeStruct((G, 1), jnp.int32),
        ),
    )(x_flat, Wr, br.reshape(1, E))

    pos_flat = pos.reshape(N)
    xs = _sc_permute_rows(x_flat, pos_flat, invert=False)

    KF = 1
    FK = FF // KF
    grid_spec = pltpu.PrefetchScalarGridSpec(
        num_scalar_prefetch=3,
        grid=(G, KF),
        in_specs=[
            pl.BlockSpec((T, D), lambda g, k, blk, eid, off: (blk[g], 0)),
            pl.BlockSpec((1, FK, D),
                         lambda g, k, blk, eid, off: (eid[g], k, 0)),
            pl.BlockSpec((1, 1, FK),
                         lambda g, k, blk, eid, off: (eid[g], 0, k)),
            pl.BlockSpec((1, D, FK),
                         lambda g, k, blk, eid, off: (eid[g], 0, k)),
            pl.BlockSpec((1, 1, D),
                         lambda g, k, blk, eid, off: (eid[g], 0, 0)),
        ],
        out_specs=pl.BlockSpec((T, D), lambda g, k, blk, eid, off: (blk[g], 0)),
    )
    ys = pl.pallas_call(
        functools.partial(_ffn_body, T=T),
        grid_spec=grid_spec,
        out_shape=jax.ShapeDtypeStruct((N, D), jnp.float32),
        compiler_params=pltpu.CompilerParams(
            dimension_semantics=("arbitrary", "arbitrary")),
    )(blkg.reshape(G), eidg.reshape(G), off.reshape(E + 1),
      xs, W1, b1.reshape(E, 1, FF), W2, b2.reshape(E, 1, D))

    out = _sc_permute_rows(ys, pos_flat, invert=True)

    return out.reshape(B, S, D)
